# Initial kernel scaffold; baseline (speedup 1.0000x reference)
import jax
import jax.numpy as jnp
from jax.experimental import pallas as pl


def kernel(x, edge_index, W1_l, b1_l, W1_r, W2_l, b2_l, W2_r):
    raise NotImplementedError("write your pallas kernel here")


# speedup vs baseline: 5.5256x; 5.5256x over previous
import functools

import jax
import jax.numpy as jnp
from jax import lax
from jax.experimental import pallas as pl
from jax.experimental.pallas import tpu as pltpu
from jax.experimental.pallas import tpu_sc as plsc

N_NODES = 10000
N_PAD = 10240
N_EDGES = 320000
D = 128
NC = 2
NS = 16
NW = NC * NS
E_PER_W = N_EDGES // NW
K = 80
N_CHUNKS = E_PER_W // K
ROWS_PER_TILE = N_PAD // NS


def _seg_sum_kernel(with_counts):
    mesh = plsc.VectorSubcoreMesh(core_axis_name="c", subcore_axis_name="s")
    out_type = [jax.ShapeDtypeStruct((NC, N_PAD, D), jnp.float32)]
    if with_counts:
        out_type.append(jax.ShapeDtypeStruct((NC, N_PAD), jnp.float32))
    scratch = [
        pltpu.VMEM_SHARED((N_PAD, D), jnp.float32),
        pltpu.VMEM((K,), jnp.int32),
        pltpu.VMEM((K,), jnp.int32),
        pltpu.VMEM((K, D), jnp.float32),
        pltpu.SemaphoreType.DMA,
    ]
    if with_counts:
        scratch += [
            pltpu.VMEM_SHARED((N_PAD,), jnp.float32),
            pltpu.VMEM((K,), jnp.float32),
        ]

    @functools.partial(pl.kernel, mesh=mesh, out_type=out_type,
                       scratch_types=scratch)
    def k(x_hbm, src_hbm, dst_hbm, zrow_hbm, zcnt_hbm, *rest):
        if with_counts:
            (out_hbm, cnt_hbm, acc, src_v, dst_v, rows_v, sem,
             cnt_acc, ones_v) = rest
        else:
            out_hbm, acc, src_v, dst_v, rows_v, sem = rest
        c = lax.axis_index("c")
        s = lax.axis_index("s")
        r0 = s * ROWS_PER_TILE
        pltpu.sync_copy(zrow_hbm, acc.at[pl.ds(r0, ROWS_PER_TILE)])
        if with_counts:
            pltpu.sync_copy(zcnt_hbm, cnt_acc.at[pl.ds(r0, ROWS_PER_TILE)])
            for j in range(K // 16):
                ones_v[pl.ds(j * 16, 16)] = jnp.ones((16,), jnp.float32)
        plsc.subcore_barrier()

        ebase = (c * NS + s) * E_PER_W

        def body(i, carry):
            base = ebase + i * K
            pltpu.sync_copy(src_hbm.at[pl.ds(base, K)], src_v)
            pltpu.sync_copy(dst_hbm.at[pl.ds(base, K)], dst_v)
            pltpu.async_copy(x_hbm.at[src_v], rows_v, sem).wait()
            pltpu.sync_copy(rows_v, acc.at[dst_v], add=True)
            if with_counts:
                pltpu.sync_copy(ones_v, cnt_acc.at[dst_v], add=True)
            return carry

        lax.fori_loop(0, N_CHUNKS, body, 0)
        plsc.subcore_barrier()
        pltpu.sync_copy(acc.at[pl.ds(r0, ROWS_PER_TILE)],
                        out_hbm.at[c, pl.ds(r0, ROWS_PER_TILE)])
        if with_counts:
            pltpu.sync_copy(cnt_acc.at[pl.ds(r0, ROWS_PER_TILE)],
                            cnt_hbm.at[c, pl.ds(r0, ROWS_PER_TILE)])

    return k


_seg_sum_cnt = _seg_sum_kernel(True)
_seg_sum = _seg_sum_kernel(False)

_BLK = 1024


def _dense_body(relu, sum_ref, cnt_ref, x_ref, wl_ref, b_ref, wr_ref, o_ref):
    cnt = cnt_ref[0, :] + cnt_ref[1, :]
    ssum = sum_ref[0] + sum_ref[1]
    mean = ssum * (1.0 / jnp.maximum(cnt, 1.0))[:, None]
    dn = (((1,), (1,)), ((), ()))
    y = (lax.dot_general(mean, wl_ref[:], dn,
                         preferred_element_type=jnp.float32)
         + b_ref[0, :]
         + lax.dot_general(x_ref[:], wr_ref[:], dn,
                           preferred_element_type=jnp.float32))
    o_ref[:] = jnp.maximum(y, 0.0) if relu else y


def _dense(summed, cnt, x, wl, b, wr, relu):
    grid = N_PAD // _BLK
    return pl.pallas_call(
        functools.partial(_dense_body, relu),
        grid=(grid,),
        in_specs=[
            pl.BlockSpec((NC, _BLK, D), lambda i: (0, i, 0)),
            pl.BlockSpec((NC, _BLK), lambda i: (0, i)),
            pl.BlockSpec((_BLK, D), lambda i: (i, 0)),
            pl.BlockSpec((D, D), lambda i: (0, 0)),
            pl.BlockSpec((1, D), lambda i: (0, 0)),
            pl.BlockSpec((D, D), lambda i: (0, 0)),
        ],
        out_specs=pl.BlockSpec((_BLK, D), lambda i: (i, 0)),
        out_shape=jax.ShapeDtypeStruct((N_PAD, D), jnp.float32),
    )(summed, cnt, x, wl, b, wr)


def kernel(x, edge_index, W1_l, b1_l, W1_r, W2_l, b2_l, W2_r):
    src = edge_index[0].astype(jnp.int32)
    dst = edge_index[1].astype(jnp.int32)
    x_pad = jnp.pad(x, ((0, N_PAD - N_NODES), (0, 0)))
    zrow = jnp.zeros((ROWS_PER_TILE, D), jnp.float32)
    zcnt = jnp.zeros((ROWS_PER_TILE,), jnp.float32)
    b1 = b1_l.reshape(1, D)
    b2 = b2_l.reshape(1, D)

    summed1, cnt = _seg_sum_cnt(x_pad, src, dst, zrow, zcnt)
    h = _dense(summed1, cnt, x_pad, W1_l, b1, W1_r, relu=True)
    (summed2,) = _seg_sum(h, src, dst, zrow, zcnt)
    out = _dense(summed2, cnt, h, W2_l, b2, W2_r, relu=False)
    return out[:N_NODES]
